# baseline (speedup 1.0000x reference)
import functools

import jax
import jax.numpy as jnp
from jax import lax
from jax.experimental import pallas as pl
from jax.experimental.pallas import tpu as pltpu
from jax.experimental.pallas import tpu_sc as plsc

N_EDGES = 320000
N_NODES = 10000
D = 128

NC = 2
NS = 16
NT = NC * NS
RPT = 320
RPC = NS * RPT
GARBAGE = RPC
ACC_ROWS = RPC + 8
CHUNK = 256
NCH16 = N_EDGES // 16


def _sc_kernel(src_hbm, idx_hbm, out_in_hbm, out_hbm,
               src_buf, idx_buf, probe_a, probe_b, acc_sh,
               sem, sem_sc, sem_init, sem_probe):
    c = lax.axis_index("c")
    s = lax.axis_index("s")
    n0 = (c * NS + s) * RPT
    r0 = s * RPT
    n_rows = jnp.minimum(jnp.int32(N_NODES) - n0, RPT)
    nbase = c * RPC
    g_row = jnp.int32(GARBAGE) + (s % 8)

    pltpu.async_copy(out_in_hbm.at[pl.ds(n0, n_rows)],
                     acc_sh.at[pl.ds(r0, n_rows)], sem_init)

    ta = n0
    tb = n0 + RPT

    def bs_body(i, st):
        lo_a, hi_a, lo_b, hi_b = st
        mid_a = (lo_a + hi_a) // 2
        mid_b = (lo_b + hi_b) // 2
        pltpu.async_copy(idx_hbm.at[pl.ds(mid_a * 16, 16)], probe_a, sem_probe)
        pltpu.async_copy(idx_hbm.at[pl.ds(mid_b * 16, 16)], probe_b, sem_probe)
        pltpu.make_async_copy(idx_hbm.at[pl.ds(mid_a * 16, 16)], probe_a,
                              sem_probe).wait()
        pltpu.make_async_copy(idx_hbm.at[pl.ds(mid_b * 16, 16)], probe_b,
                              sem_probe).wait()
        take_a = probe_a[...][0] < ta
        take_b = probe_b[...][0] < tb
        return (jnp.where(take_a, mid_a, lo_a), jnp.where(take_a, hi_a, mid_a),
                jnp.where(take_b, mid_b, lo_b), jnp.where(take_b, hi_b, mid_b))

    lo_a, _, lo_b, _ = lax.fori_loop(
        0, 15, bs_body,
        (jnp.int32(0), jnp.int32(NCH16), jnp.int32(0), jnp.int32(NCH16)))
    pltpu.async_copy(idx_hbm.at[pl.ds(lo_a * 16, 16)], probe_a, sem_probe)
    pltpu.async_copy(idx_hbm.at[pl.ds(lo_b * 16, 16)], probe_b, sem_probe)
    pltpu.make_async_copy(idx_hbm.at[pl.ds(lo_a * 16, 16)], probe_a,
                          sem_probe).wait()
    pltpu.make_async_copy(idx_hbm.at[pl.ds(lo_b * 16, 16)], probe_b,
                          sem_probe).wait()
    pa = probe_a[...]
    pb = probe_b[...]
    below_a = jnp.int32(0)
    below_b = jnp.int32(0)
    for k in range(16):
        below_a = below_a + jnp.where(pa[k] < ta, 1, 0)
        below_b = below_b + jnp.where(pb[k] < tb, 1, 0)
    my_lo = lo_a * 16 + below_a
    my_hi = lo_b * 16 + below_b

    pltpu.make_async_copy(out_in_hbm.at[pl.ds(n0, n_rows)],
                          acc_sh.at[pl.ds(r0, n_rows)], sem_init).wait()

    a0 = (my_lo // 8) * 8
    n_chunks = (my_hi - a0 + CHUNK - 1) // CHUNK

    def start_loads(t, b):
        e_c = jnp.minimum(a0 + t * CHUNK, N_EDGES - CHUNK)
        pltpu.async_copy(src_hbm.at[pl.ds(e_c, CHUNK), :], src_buf.at[b], sem)
        for h in range(CHUNK // 128):
            pltpu.async_copy(idx_hbm.at[pl.ds(e_c + h * 128, 128)],
                             idx_buf.at[b, h], sem)

    def wait_loads(t, b):
        e_c = jnp.minimum(a0 + t * CHUNK, N_EDGES - CHUNK)
        pltpu.make_async_copy(src_hbm.at[pl.ds(e_c, CHUNK), :],
                              src_buf.at[b], sem).wait()
        for h in range(CHUNK // 128):
            pltpu.make_async_copy(idx_hbm.at[pl.ds(e_c + h * 128, 128)],
                                  idx_buf.at[b, h], sem).wait()

    @pl.when(n_chunks > 0)
    def _():
        start_loads(0, 0)

    iota16 = lax.iota(jnp.int32, 16)

    def start_scatter(b):
        for h in range(CHUNK // 128):
            pltpu.async_copy(src_buf.at[b, pl.ds(h * 128, 128), :],
                             acc_sh.at[idx_buf.at[b, h]], sem_sc, add=True)

    def wait_scatter(b):
        for h in range(CHUNK // 128):
            pltpu.make_async_copy(src_buf.at[b, pl.ds(h * 128, 128), :],
                                  acc_sh.at[idx_buf.at[b, h]], sem_sc).wait()

    def chunk_body(t, _):
        b = t % 2
        wait_loads(t, b)

        @pl.when(t >= 1)
        def _():
            wait_scatter(1 - b)

        @pl.when(t + 1 < n_chunks)
        def _():
            start_loads(t + 1, 1 - b)

        e_c = jnp.minimum(a0 + t * CHUNK, N_EDGES - CHUNK)
        lmax = jnp.maximum(my_lo, a0 + t * CHUNK)
        interior = jnp.logical_and(lmax <= e_c, e_c + CHUNK <= my_hi)

        @pl.when(jnp.logical_not(jnp.logical_and(interior, nbase == 0)))
        def _():
            for h in range(CHUNK // 128):
                for k in range(8):
                    pos = e_c + h * 128 + k * 16 + iota16
                    v = idx_buf[b, h, pl.ds(k * 16, 16)]
                    ok = jnp.logical_and(pos >= lmax, pos < my_hi)
                    idx_buf[b, h, pl.ds(k * 16, 16)] = jnp.where(
                        ok, v - nbase, g_row)

        start_scatter(b)
        return 0

    lax.fori_loop(0, n_chunks, chunk_body, 0)

    @pl.when(n_chunks > 0)
    def _():
        wait_scatter((n_chunks - 1) % 2)

    pltpu.sync_copy(acc_sh.at[pl.ds(r0, n_rows)], out_hbm.at[pl.ds(n0, n_rows)])


@functools.partial(
    pl.kernel,
    mesh=plsc.VectorSubcoreMesh(core_axis_name="c", subcore_axis_name="s"),
    out_type=jax.ShapeDtypeStruct((N_NODES, D), jnp.float32),
    scratch_types=[
        pltpu.VMEM((2, CHUNK, D), jnp.float32),
        pltpu.VMEM((2, CHUNK // 128, 128), jnp.int32),
        pltpu.VMEM((16,), jnp.int32),
        pltpu.VMEM((16,), jnp.int32),
        pltpu.VMEM_SHARED((ACC_ROWS, D), jnp.float32),
        pltpu.SemaphoreType.DMA,
        pltpu.SemaphoreType.DMA,
        pltpu.SemaphoreType.DMA,
        pltpu.SemaphoreType.DMA,
    ],
)
def _sc_scatter(src_hbm, idx_hbm, out_in_hbm, out_hbm,
                src_buf, idx_buf, probe_a, probe_b, acc_sh,
                sem, sem_sc, sem_init, sem_probe):
    _sc_kernel(src_hbm, idx_hbm, out_in_hbm, out_hbm,
               src_buf, idx_buf, probe_a, probe_b, acc_sh,
               sem, sem_sc, sem_init, sem_probe)


@jax.jit
def kernel(src, index, out):
    idx = index.astype(jnp.int32)
    return _sc_scatter(src, idx, out)
